# baseline (speedup 1.0000x reference)
import functools

import jax
import jax.numpy as jnp
from jax import lax
from jax.experimental import pallas as pl
from jax.experimental.pallas import tpu as pltpu
from jax.experimental.pallas import tpu_sc as plsc

H = 128
_NNODES = {"drug": 10000, "protein": 10000, "disease": 2048}
_RELS = [
    ("e_d_t_dr", "disease", "drug"),
    ("e_d_m_dr", "disease", "drug"),
    ("e_d_p", "disease", "protein"),
    ("e_dr_t_d", "drug", "disease"),
    ("e_dr_m_d", "drug", "disease"),
    ("e_p_d", "protein", "disease"),
    ("e_DDI", "drug", "drug"),
    ("e_PPI", "protein", "protein"),
]

_NC = 2
_NS = 16
_C = 2048
_G = 32
_LANES = 16
_KB = H // _LANES


def _rup(x, m):
    return (x + m - 1) // m * m


_RPW = {nt: _rup(_NNODES[nt], _NS) // _NS for nt in _NNODES}
_RMAX = max(_RPW.values())


def _seg_body(*refs):
    m_refs = refs[0:8]
    src_refs = refs[8:16]
    dst_refs = refs[16:24]
    out_refs = refs[24:32]
    (srcA, dstA, srcB, dstB, sel_src, sel_dst) = refs[32:38]
    rows_bufs = refs[38:42]
    accs = refs[42:50]
    esemA, esemB = refs[50:52]
    gsems = refs[52:56]

    cid = lax.axis_index("c")
    sid = lax.axis_index("s")
    wid = sid * _NC + cid
    iota = lax.iota(jnp.int32, _LANES)
    zeros16 = jnp.zeros((_LANES,), jnp.float32)
    sent_src = wid * 8

    def scan_and_merge(r, R, lo, srcbuf, dstbuf):
        def scan_body(i, cnt_vec):
            for half in range(2):
                off = (2 * i + half) * _LANES
                d = dstbuf[pl.ds(off, _LANES)]
                s = srcbuf[pl.ds(off, _LANES)]
                rel = d - lo
                mask = plsc.bitcast(rel, jnp.uint32) < jnp.uint32(R)
                p = jnp.where(mask, 1, 0)
                for sh in (1, 2, 4, 8):
                    idxs = jnp.maximum(iota - sh, 0)
                    p = p + jnp.where(iota >= sh, jnp.take(p, idxs), 0)
                posn = cnt_vec + p - 1
                plsc.store_scatter(sel_src, [posn], s, mask=mask)
                plsc.store_scatter(sel_dst, [posn], rel, mask=mask)
                cnt_vec = cnt_vec + plsc.all_reduce_population_count(mask)
            return cnt_vec

        cnt_vec = lax.fori_loop(0, _C // (2 * _LANES), scan_body,
                                jnp.zeros((_LANES,), jnp.int32))

        pad_rel = jnp.full((_LANES,), R, jnp.int32)
        pad_src = jnp.full((_LANES,), sent_src, jnp.int32)
        plsc.store_scatter(sel_dst, [cnt_vec + iota], pad_rel)
        plsc.store_scatter(sel_src, [cnt_vec + iota], pad_src)
        plsc.store_scatter(sel_dst, [cnt_vec + 16 + iota], pad_rel)
        plsc.store_scatter(sel_src, [cnt_vec + 16 + iota], pad_src)
        cnt = jnp.max(cnt_vec)
        n_grp = (cnt + _G - 1) // _G

        def merge_group(grp, rows):
            def edge_body(e, _):
                e_vec = jnp.full((_LANES,), e, jnp.int32)
                de = plsc.load_gather(
                    sel_dst, [jnp.full((_LANES,), grp * _G + e, jnp.int32)])
                base = de * _LANES + iota
                rvs = [plsc.load_gather(rows, [e_vec, iota + (k * _LANES)])
                       for k in range(_KB)]
                avs = [plsc.load_gather(accs[k], [base]) for k in range(_KB)]
                for k in range(_KB):
                    plsc.store_scatter(accs[k], [base],
                                       jnp.maximum(avs[k], rvs[k]))
                return 0

            lax.fori_loop(0, _G, edge_body, 0)

        def fire(grp, rows, gsem):
            return pltpu.async_copy(
                m_refs[r].at[sel_src.at[pl.ds(grp * _G, _G)]], rows, gsem)

        n_quad = (n_grp + 3) // 4
        clamp = lambda g: jnp.minimum(g, jnp.maximum(n_grp - 1, 0))
        for j in range(4):
            fire(clamp(j), rows_bufs[j], gsems[j])

        def quad_body(q, _):
            g0 = 4 * q
            for j in range(4):
                pltpu.make_async_copy(
                    m_refs[r].at[sel_src.at[pl.ds(0, _G)]], rows_bufs[j],
                    gsems[j]).wait()
                merge_group(clamp(g0 + j), rows_bufs[j])
                fire(clamp(g0 + j + 4), rows_bufs[j], gsems[j])
            return 0

        lax.fori_loop(0, n_quad, quad_body, 0)
        for j in range(4):
            pltpu.make_async_copy(
                m_refs[r].at[sel_src.at[pl.ds(0, _G)]], rows_bufs[j],
                gsems[j]).wait()

    for r in range(8):
        _, _, dt = _RELS[r]
        R = _RPW[dt]
        lo = sid * R
        half_len = src_refs[r].shape[0] // _NC
        base_e = cid * half_len
        n_chunks = half_len // _C
        n_src = m_refs[r].shape[0]

        def zero_body(j, _):
            for k in range(_KB):
                accs[k][pl.ds(j * _LANES, _LANES)] = zeros16
            return 0
        lax.fori_loop(0, R + 1, zero_body, 0)

        def cpair_body(cp, _, r=r, R=R, lo=lo, base_e=base_e,
                       n_chunks=n_chunks):
            offA = base_e + (2 * cp) * _C
            offB = base_e + jnp.minimum(2 * cp + 1, n_chunks - 1) * _C
            cpA1 = pltpu.async_copy(src_refs[r].at[pl.ds(offA, _C)], srcA,
                                    esemA)
            cpA2 = pltpu.async_copy(dst_refs[r].at[pl.ds(offA, _C)], dstA,
                                    esemA)
            cpB1 = pltpu.async_copy(src_refs[r].at[pl.ds(offB, _C)], srcB,
                                    esemB)
            cpB2 = pltpu.async_copy(dst_refs[r].at[pl.ds(offB, _C)], dstB,
                                    esemB)
            cpA1.wait()
            cpA2.wait()
            scan_and_merge(r, R, lo, srcA, dstA)
            cpB1.wait()
            cpB2.wait()
            scan_and_merge(r, R, lo, srcB, dstB)
            return 0

        lax.fori_loop(0, (n_chunks + 1) // 2, cpair_body, 0)

        wbase = (wid * _KB) * (R * _LANES)
        for k in range(_KB):
            pltpu.sync_copy(
                accs[k].at[pl.ds(0, R * _LANES)],
                out_refs[r].at[pl.ds(wbase + k * (R * _LANES), R * _LANES)])


def _segment_max_layer(ms, srcs, dsts):
    out_types = [
        jax.ShapeDtypeStruct((_NC * _NS * _KB * _RPW[_RELS[r][2]] * _LANES,),
                             jnp.float32)
        for r in range(8)
    ]
    mesh = plsc.VectorSubcoreMesh(core_axis_name="c", subcore_axis_name="s")
    fn = pl.kernel(
        _seg_body,
        out_type=out_types,
        mesh=mesh,
        scratch_types=(
            [pltpu.VMEM((_C,), jnp.int32)] * 4
            + [pltpu.VMEM((_C + 64,), jnp.int32)] * 2
            + [pltpu.VMEM((_G, H), jnp.float32)] * 4
            + [pltpu.VMEM(((_RMAX + 1) * _LANES,), jnp.float32)] * _KB
            + [pltpu.SemaphoreType.DMA] * 6
        ),
        compiler_params=pltpu.CompilerParams(needs_layout_passes=False),
    )
    outs = fn(*ms, *srcs, *dsts)
    res = []
    for r, o in enumerate(outs):
        n_dst = _NNODES[_RELS[r][2]]
        R = _RPW[_RELS[r][2]]
        o = o.reshape(_NS, _NC, _KB, R, _LANES).transpose(1, 0, 3, 2, 4)
        o = o.reshape(_NC, _NS * R, H)
        res.append(jnp.maximum(o[0], o[1])[:n_dst])
    return res


def _bn_relu(z, g, b):
    m = jnp.mean(z, 0)
    v = jnp.mean((z - m) ** 2, 0)
    return jax.nn.relu((z - m) * lax.rsqrt(v + 1e-5) * g + b)


def _head_body(feat_ref, W1, b1, g1, be1, W2, b2, g2, be2, W3, b3, g3, be3,
               Wout, bout, out_ref):
    z = jnp.dot(feat_ref[...], W1[...], preferred_element_type=jnp.float32) + b1[...]
    o = _bn_relu(z, g1[...], be1[...])
    z = jnp.dot(o, W2[...], preferred_element_type=jnp.float32) + b2[...]
    o = _bn_relu(z, g2[...], be2[...])
    z = jnp.dot(o, W3[...], preferred_element_type=jnp.float32) + b3[...]
    o = _bn_relu(z, g3[...], be3[...])
    z = jnp.dot(o, Wout[...], preferred_element_type=jnp.float32) + bout[...]
    out_ref[...] = jax.nn.sigmoid(z)


def _head(feat, W1, b1, g1, be1, W2, b2, g2, be2, W3, b3, g3, be3, Wout, bout):
    B = feat.shape[0]
    return pl.pallas_call(
        _head_body,
        out_shape=jax.ShapeDtypeStruct((B, 1), jnp.float32),
    )(feat, W1, b1, g1, be1, W2, b2, g2, be2, W3, b3, g3, be3, Wout, bout)


def _hgcn(h, srcs, dsts, Wp, bp, Ws, Wn, bs):
    ms = []
    for i, (name, st, dt) in enumerate(_RELS):
        ms.append(jax.nn.relu(h[st] @ Wp[i] + bp[i]))
    neighs = _segment_max_layer(ms, srcs, dsts)
    out = {nt: jnp.zeros((_NNODES[nt], H), jnp.float32) for nt in _NNODES}
    for i, (name, st, dt) in enumerate(_RELS):
        out[dt] = out[dt] + jax.nn.relu(
            h[dt] @ Ws[i] + neighs[i] @ Wn[i] + bs[i])
    return out


def kernel(x_dr, x_p, finger_feats, seq_feats, disease_feat, e_d_t_dr, e_d_m_dr,
           e_d_p, e_dr_t_d, e_dr_m_d, e_p_d, e_DDI, e_PPI, W_fing, b_fing,
           W_seq, b_seq, W_dis, b_dis, Wp, bp, Ws, Wn, bs, W1, b1, g1, be1,
           W2, b2, g2, be2, W3, b3, g3, be3, Wout, bout):
    edges = [e_d_t_dr, e_d_m_dr, e_d_p, e_dr_t_d, e_dr_m_d, e_p_d, e_DDI,
             e_PPI]
    srcs, dsts = [], []
    for e in edges:
        ne = e.shape[1]
        half = ne // _NC
        pad = _rup(half, 2 * _C) - half
        s2 = e[0].reshape(_NC, half).astype(jnp.int32)
        d2 = e[1].reshape(_NC, half).astype(jnp.int32)
        srcs.append(jnp.pad(s2, ((0, 0), (0, pad))).reshape(-1))
        dsts.append(jnp.pad(d2, ((0, 0), (0, pad)),
                            constant_values=-1).reshape(-1))

    h_dr_f = jax.nn.relu(finger_feats @ W_fing + b_fing)
    h_p_s = jax.nn.relu(seq_feats @ W_seq + b_seq)
    h_d = jax.nn.relu(disease_feat @ W_dis + b_dis)
    h0 = {"drug": h_dr_f, "protein": h_p_s, "disease": h_d}
    h1 = _hgcn(h0, srcs, dsts, Wp, bp, Ws, Wn, bs)
    h2 = _hgcn(h1, srcs, dsts, Wp, bp, Ws, Wn, bs)
    dr_new = jnp.concatenate([h_dr_f, h1["drug"], h2["drug"]], axis=1)
    p_new = jnp.concatenate([h_p_s, h1["protein"], h2["protein"]], axis=1)
    feat = jnp.concatenate([dr_new[x_dr], p_new[x_p]], axis=1)
    return _head(feat, W1, b1, g1, be1, W2, b2, g2, be2, W3, b3, g3, be3,
                 Wout, bout)


# speedup vs baseline: 1.0858x; 1.0858x over previous
import functools

import jax
import jax.numpy as jnp
from jax import lax
from jax.experimental import pallas as pl
from jax.experimental.pallas import tpu as pltpu
from jax.experimental.pallas import tpu_sc as plsc

H = 128
_NNODES = {"drug": 10000, "protein": 10000, "disease": 2048}
_RELS = [
    ("e_d_t_dr", "disease", "drug"),
    ("e_d_m_dr", "disease", "drug"),
    ("e_d_p", "disease", "protein"),
    ("e_dr_t_d", "drug", "disease"),
    ("e_dr_m_d", "drug", "disease"),
    ("e_p_d", "protein", "disease"),
    ("e_DDI", "drug", "drug"),
    ("e_PPI", "protein", "protein"),
]

_NC = 2
_NS = 16
_C = 2048
_G = 32
_LANES = 16
_KB = H // _LANES


def _rup(x, m):
    return (x + m - 1) // m * m


_RPW = {nt: _rup(_NNODES[nt], _NS) // _NS for nt in _NNODES}
_RMAX = max(_RPW.values())


def _seg_body(*refs):
    m_refs = refs[0:8]
    src_refs = refs[8:16]
    dst_refs = refs[16:24]
    out_refs = refs[24:32]
    (srcA, dstA, srcB, dstB, sel_src, sel_dst) = refs[32:38]
    rows_bufs = refs[38:40]
    accs = refs[40:48]
    esemA, esemB = refs[48:50]
    gsems = refs[50:52]

    cid = lax.axis_index("c")
    sid = lax.axis_index("s")
    wid = sid * _NC + cid
    iota = lax.iota(jnp.int32, _LANES)
    zeros16 = jnp.zeros((_LANES,), jnp.float32)
    sent_src = wid * 8

    def scan_and_merge(r, R, lo, srcbuf, dstbuf):
        def scan_body(i, cnt_vec):
            for half in range(2):
                off = (2 * i + half) * _LANES
                d = dstbuf[pl.ds(off, _LANES)]
                s = srcbuf[pl.ds(off, _LANES)]
                rel = d - lo
                mask = plsc.bitcast(rel, jnp.uint32) < jnp.uint32(R)
                p = jnp.where(mask, 1, 0)
                for sh in (1, 2, 4, 8):
                    idxs = jnp.maximum(iota - sh, 0)
                    p = p + jnp.where(iota >= sh, jnp.take(p, idxs), 0)
                posn = cnt_vec + p - 1
                plsc.store_scatter(sel_src, [posn], s, mask=mask)
                plsc.store_scatter(sel_dst, [posn], rel, mask=mask)
                cnt_vec = cnt_vec + plsc.all_reduce_population_count(mask)
            return cnt_vec

        cnt_vec = lax.fori_loop(0, _C // (2 * _LANES), scan_body,
                                jnp.zeros((_LANES,), jnp.int32))

        pad_rel = jnp.full((_LANES,), R, jnp.int32)
        pad_src = jnp.full((_LANES,), sent_src, jnp.int32)
        plsc.store_scatter(sel_dst, [cnt_vec + iota], pad_rel)
        plsc.store_scatter(sel_src, [cnt_vec + iota], pad_src)
        plsc.store_scatter(sel_dst, [cnt_vec + 16 + iota], pad_rel)
        plsc.store_scatter(sel_src, [cnt_vec + 16 + iota], pad_src)
        cnt = jnp.max(cnt_vec)
        n_grp = (cnt + _G - 1) // _G

        def merge_group(grp, rows):
            def edge_body(e, _):
                e_vec = jnp.full((_LANES,), e, jnp.int32)
                de = plsc.load_gather(
                    sel_dst, [jnp.full((_LANES,), grp * _G + e, jnp.int32)])
                base = de * _LANES + iota
                rvs = [plsc.load_gather(rows, [e_vec, iota + (k * _LANES)])
                       for k in range(_KB)]
                avs = [plsc.load_gather(accs[k], [base]) for k in range(_KB)]
                for k in range(_KB):
                    plsc.store_scatter(accs[k], [base],
                                       jnp.maximum(avs[k], rvs[k]))
                return 0

            lax.fori_loop(0, _G, edge_body, 0)

        def fire(grp, rows, gsem):
            return pltpu.async_copy(
                m_refs[r].at[sel_src.at[pl.ds(grp * _G, _G)]], rows, gsem)

        n_pair = (n_grp + 1) // 2
        clamp = lambda g: jnp.minimum(g, jnp.maximum(n_grp - 1, 0))
        for j in range(2):
            fire(clamp(j), rows_bufs[j], gsems[j])

        def gpair_body(q, _):
            g0 = 2 * q
            for j in range(2):
                pltpu.make_async_copy(
                    m_refs[r].at[sel_src.at[pl.ds(0, _G)]], rows_bufs[j],
                    gsems[j]).wait()
                merge_group(clamp(g0 + j), rows_bufs[j])
                fire(clamp(g0 + j + 2), rows_bufs[j], gsems[j])
            return 0

        lax.fori_loop(0, n_pair, gpair_body, 0)
        for j in range(2):
            pltpu.make_async_copy(
                m_refs[r].at[sel_src.at[pl.ds(0, _G)]], rows_bufs[j],
                gsems[j]).wait()

    for r in range(8):
        _, _, dt = _RELS[r]
        R = _RPW[dt]
        lo = sid * R
        half_len = src_refs[r].shape[0] // _NC
        base_e = cid * half_len
        n_chunks = half_len // _C
        n_src = m_refs[r].shape[0]

        def zero_body(j, _):
            for k in range(_KB):
                accs[k][pl.ds(j * _LANES, _LANES)] = zeros16
            return 0
        lax.fori_loop(0, R + 1, zero_body, 0)

        def cpair_body(cp, _, r=r, R=R, lo=lo, base_e=base_e,
                       n_chunks=n_chunks):
            offA = base_e + (2 * cp) * _C
            offB = base_e + jnp.minimum(2 * cp + 1, n_chunks - 1) * _C
            cpA1 = pltpu.async_copy(src_refs[r].at[pl.ds(offA, _C)], srcA,
                                    esemA)
            cpA2 = pltpu.async_copy(dst_refs[r].at[pl.ds(offA, _C)], dstA,
                                    esemA)
            cpB1 = pltpu.async_copy(src_refs[r].at[pl.ds(offB, _C)], srcB,
                                    esemB)
            cpB2 = pltpu.async_copy(dst_refs[r].at[pl.ds(offB, _C)], dstB,
                                    esemB)
            cpA1.wait()
            cpA2.wait()
            scan_and_merge(r, R, lo, srcA, dstA)
            cpB1.wait()
            cpB2.wait()
            scan_and_merge(r, R, lo, srcB, dstB)
            return 0

        lax.fori_loop(0, (n_chunks + 1) // 2, cpair_body, 0)

        wbase = (wid * _KB) * (R * _LANES)
        for k in range(_KB):
            pltpu.sync_copy(
                accs[k].at[pl.ds(0, R * _LANES)],
                out_refs[r].at[pl.ds(wbase + k * (R * _LANES), R * _LANES)])


def _segment_max_layer(ms, srcs, dsts):
    out_types = [
        jax.ShapeDtypeStruct((_NC * _NS * _KB * _RPW[_RELS[r][2]] * _LANES,),
                             jnp.float32)
        for r in range(8)
    ]
    mesh = plsc.VectorSubcoreMesh(core_axis_name="c", subcore_axis_name="s")
    fn = pl.kernel(
        _seg_body,
        out_type=out_types,
        mesh=mesh,
        scratch_types=(
            [pltpu.VMEM((_C,), jnp.int32)] * 4
            + [pltpu.VMEM((_C + 64,), jnp.int32)] * 2
            + [pltpu.VMEM((_G, H), jnp.float32)] * 2
            + [pltpu.VMEM(((_RMAX + 1) * _LANES,), jnp.float32)] * _KB
            + [pltpu.SemaphoreType.DMA] * 4
        ),
        compiler_params=pltpu.CompilerParams(needs_layout_passes=False),
    )
    outs = fn(*ms, *srcs, *dsts)
    res = []
    for r, o in enumerate(outs):
        n_dst = _NNODES[_RELS[r][2]]
        R = _RPW[_RELS[r][2]]
        o = o.reshape(_NS, _NC, _KB, R, _LANES).transpose(1, 0, 3, 2, 4)
        o = o.reshape(_NC, _NS * R, H)
        res.append(jnp.maximum(o[0], o[1])[:n_dst])
    return res


def _mm_relu_body(x_ref, w_ref, b_ref, o_ref):
    o_ref[...] = jax.nn.relu(
        jnp.dot(x_ref[...], w_ref[...], preferred_element_type=jnp.float32)
        + b_ref[...])


def _mm_relu(x, w, b, block_rows=0):
    n, k = x.shape
    ho = w.shape[1]
    if not block_rows or block_rows >= n:
        return pl.pallas_call(
            _mm_relu_body,
            out_shape=jax.ShapeDtypeStruct((n, ho), jnp.float32),
        )(x, w, b)
    assert n % block_rows == 0
    return pl.pallas_call(
        _mm_relu_body,
        grid=(n // block_rows,),
        in_specs=[
            pl.BlockSpec((block_rows, k), lambda i: (i, 0)),
            pl.BlockSpec((k, ho), lambda i: (0, 0)),
            pl.BlockSpec((ho,), lambda i: (0,)),
        ],
        out_specs=pl.BlockSpec((block_rows, ho), lambda i: (i, 0)),
        out_shape=jax.ShapeDtypeStruct((n, ho), jnp.float32),
    )(x, w, b)


def _sage_out_body(h_ref, n_ref, ws_ref, wn_ref, b_ref, o_ref):
    acc = None
    for j in range(n_ref.shape[0]):
        t = jax.nn.relu(
            jnp.dot(h_ref[...], ws_ref[j], preferred_element_type=jnp.float32)
            + jnp.dot(n_ref[j], wn_ref[j], preferred_element_type=jnp.float32)
            + b_ref[j])
        acc = t if acc is None else acc + t
    o_ref[...] = acc


def _sage_out(h_dt, neighs, Ws_sel, Wn_sel, bs_sel):
    n = h_dt.shape[0]
    return pl.pallas_call(
        _sage_out_body,
        out_shape=jax.ShapeDtypeStruct((n, H), jnp.float32),
    )(h_dt, jnp.stack(neighs), Ws_sel, Wn_sel, bs_sel)


def _bn_relu(z, g, b):
    m = jnp.mean(z, 0)
    v = jnp.mean((z - m) ** 2, 0)
    return jax.nn.relu((z - m) * lax.rsqrt(v + 1e-5) * g + b)


def _head_body(feat_ref, W1, b1, g1, be1, W2, b2, g2, be2, W3, b3, g3, be3,
               Wout, bout, out_ref):
    z = jnp.dot(feat_ref[...], W1[...], preferred_element_type=jnp.float32) + b1[...]
    o = _bn_relu(z, g1[...], be1[...])
    z = jnp.dot(o, W2[...], preferred_element_type=jnp.float32) + b2[...]
    o = _bn_relu(z, g2[...], be2[...])
    z = jnp.dot(o, W3[...], preferred_element_type=jnp.float32) + b3[...]
    o = _bn_relu(z, g3[...], be3[...])
    z = jnp.dot(o, Wout[...], preferred_element_type=jnp.float32) + bout[...]
    out_ref[...] = jax.nn.sigmoid(z)


def _head(feat, W1, b1, g1, be1, W2, b2, g2, be2, W3, b3, g3, be3, Wout, bout):
    B = feat.shape[0]
    return pl.pallas_call(
        _head_body,
        out_shape=jax.ShapeDtypeStruct((B, 1), jnp.float32),
    )(feat, W1, b1, g1, be1, W2, b2, g2, be2, W3, b3, g3, be3, Wout, bout)


_SRC_GROUPS = {"disease": [0, 1, 2], "drug": [3, 4, 6], "protein": [5, 7]}
_DST_GROUPS = {"drug": [0, 1, 6], "protein": [2, 7], "disease": [3, 4, 5]}


def _hgcn(h, srcs, dsts, Wp, bp, Ws, Wn, bs):
    ms = [None] * 8
    for st, ids in _SRC_GROUPS.items():
        wcat = jnp.concatenate([Wp[i] for i in ids], axis=1)
        bcat = jnp.concatenate([bp[i] for i in ids])
        mcat = _mm_relu(h[st], wcat, bcat)
        for j, i in enumerate(ids):
            ms[i] = mcat[:, j * H:(j + 1) * H]
    neighs = _segment_max_layer(ms, srcs, dsts)
    out = {}
    for dt, ids in _DST_GROUPS.items():
        out[dt] = _sage_out(h[dt], [neighs[i] for i in ids],
                            Ws[jnp.array(ids)], Wn[jnp.array(ids)],
                            bs[jnp.array(ids)])
    return out


def kernel(x_dr, x_p, finger_feats, seq_feats, disease_feat, e_d_t_dr, e_d_m_dr,
           e_d_p, e_dr_t_d, e_dr_m_d, e_p_d, e_DDI, e_PPI, W_fing, b_fing,
           W_seq, b_seq, W_dis, b_dis, Wp, bp, Ws, Wn, bs, W1, b1, g1, be1,
           W2, b2, g2, be2, W3, b3, g3, be3, Wout, bout):
    edges = [e_d_t_dr, e_d_m_dr, e_d_p, e_dr_t_d, e_dr_m_d, e_p_d, e_DDI,
             e_PPI]
    srcs, dsts = [], []
    for e in edges:
        ne = e.shape[1]
        half = ne // _NC
        pad = _rup(half, 2 * _C) - half
        s2 = e[0].reshape(_NC, half).astype(jnp.int32)
        d2 = e[1].reshape(_NC, half).astype(jnp.int32)
        srcs.append(jnp.pad(s2, ((0, 0), (0, pad))).reshape(-1))
        dsts.append(jnp.pad(d2, ((0, 0), (0, pad)),
                            constant_values=-1).reshape(-1))

    h_dr_f = _mm_relu(finger_feats, W_fing, b_fing, block_rows=2000)
    h_p_s = _mm_relu(seq_feats, W_seq, b_seq, block_rows=2000)
    h_d = _mm_relu(disease_feat, W_dis, b_dis)
    h0 = {"drug": h_dr_f, "protein": h_p_s, "disease": h_d}
    h1 = _hgcn(h0, srcs, dsts, Wp, bp, Ws, Wn, bs)
    h2 = _hgcn(h1, srcs, dsts, Wp, bp, Ws, Wn, bs)
    dr_new = jnp.concatenate([h_dr_f, h1["drug"], h2["drug"]], axis=1)
    p_new = jnp.concatenate([h_p_s, h1["protein"], h2["protein"]], axis=1)
    feat = jnp.concatenate([dr_new[x_dr], p_new[x_p]], axis=1)
    return _head(feat, W1, b1, g1, be1, W2, b2, g2, be2, W3, b3, g3, be3,
                 Wout, bout)


# speedup vs baseline: 1.2486x; 1.1499x over previous
import functools

import jax
import jax.numpy as jnp
from jax import lax
from jax.experimental import pallas as pl
from jax.experimental.pallas import tpu as pltpu
from jax.experimental.pallas import tpu_sc as plsc

H = 128
_NNODES = {"drug": 10000, "protein": 10000, "disease": 2048}
_RELS = [
    ("e_d_t_dr", "disease", "drug"),
    ("e_d_m_dr", "disease", "drug"),
    ("e_d_p", "disease", "protein"),
    ("e_dr_t_d", "drug", "disease"),
    ("e_dr_m_d", "drug", "disease"),
    ("e_p_d", "protein", "disease"),
    ("e_DDI", "drug", "drug"),
    ("e_PPI", "protein", "protein"),
]

_NC = 2
_NS = 16
_C = 4096
_G = 32
_LANES = 16
_KB = H // _LANES


def _rup(x, m):
    return (x + m - 1) // m * m


_RPW = {nt: _rup(_NNODES[nt], _NS) // _NS for nt in _NNODES}
_RMAX = max(_RPW.values())


def _seg_body(*refs):
    m_refs = refs[0:8]
    src_refs = refs[8:16]
    dst_refs = refs[16:24]
    out_refs = refs[24:32]
    (srcA, dstA, srcB, dstB, sel_src, sel_dst) = refs[32:38]
    rows_bufs = refs[38:40]
    accs = refs[40:48]
    esemA, esemB = refs[48:50]
    gsems = refs[50:52]

    cid = lax.axis_index("c")
    sid = lax.axis_index("s")
    wid = sid * _NC + cid
    iota = lax.iota(jnp.int32, _LANES)
    zeros16 = jnp.zeros((_LANES,), jnp.float32)
    sent_src = wid * 8

    def scan_and_merge(r, R, lo, srcbuf, dstbuf):
        def scan_body(i, cnt_vec):
            for half in range(2):
                off = (2 * i + half) * _LANES
                d = dstbuf[pl.ds(off, _LANES)]
                s = srcbuf[pl.ds(off, _LANES)]
                rel = d - lo
                mask = plsc.bitcast(rel, jnp.uint32) < jnp.uint32(R)
                p = jnp.where(mask, 1, 0)
                for sh in (1, 2, 4, 8):
                    idxs = jnp.maximum(iota - sh, 0)
                    p = p + jnp.where(iota >= sh, jnp.take(p, idxs), 0)
                posn = cnt_vec + p - 1
                plsc.store_scatter(sel_src, [posn], s, mask=mask)
                plsc.store_scatter(sel_dst, [posn], rel, mask=mask)
                cnt_vec = cnt_vec + plsc.all_reduce_population_count(mask)
            return cnt_vec

        cnt_vec = lax.fori_loop(0, _C // (2 * _LANES), scan_body,
                                jnp.zeros((_LANES,), jnp.int32))

        pad_rel = jnp.full((_LANES,), R, jnp.int32)
        pad_src = jnp.full((_LANES,), sent_src, jnp.int32)
        plsc.store_scatter(sel_dst, [cnt_vec + iota], pad_rel)
        plsc.store_scatter(sel_src, [cnt_vec + iota], pad_src)
        plsc.store_scatter(sel_dst, [cnt_vec + 16 + iota], pad_rel)
        plsc.store_scatter(sel_src, [cnt_vec + 16 + iota], pad_src)
        cnt = jnp.max(cnt_vec)
        n_grp = (cnt + _G - 1) // _G

        def merge_group(grp, rows):
            def edge_body(e, _):
                e_vec = jnp.full((_LANES,), e, jnp.int32)
                de = plsc.load_gather(
                    sel_dst, [jnp.full((_LANES,), grp * _G + e, jnp.int32)])
                base = de * _LANES + iota
                rvs = [plsc.load_gather(rows, [e_vec, iota + (k * _LANES)])
                       for k in range(_KB)]
                avs = [plsc.load_gather(accs[k], [base]) for k in range(_KB)]
                for k in range(_KB):
                    plsc.store_scatter(accs[k], [base],
                                       jnp.maximum(avs[k], rvs[k]))
                return 0

            lax.fori_loop(0, _G, edge_body, 0)

        def fire(grp, rows, gsem):
            return pltpu.async_copy(
                m_refs[r].at[sel_src.at[pl.ds(grp * _G, _G)]], rows, gsem)

        n_pair = (n_grp + 1) // 2
        clamp = lambda g: jnp.minimum(g, jnp.maximum(n_grp - 1, 0))
        for j in range(2):
            fire(clamp(j), rows_bufs[j], gsems[j])

        def gpair_body(q, _):
            g0 = 2 * q
            for j in range(2):
                pltpu.make_async_copy(
                    m_refs[r].at[sel_src.at[pl.ds(0, _G)]], rows_bufs[j],
                    gsems[j]).wait()
                merge_group(clamp(g0 + j), rows_bufs[j])
                fire(clamp(g0 + j + 2), rows_bufs[j], gsems[j])
            return 0

        lax.fori_loop(0, n_pair, gpair_body, 0)
        for j in range(2):
            pltpu.make_async_copy(
                m_refs[r].at[sel_src.at[pl.ds(0, _G)]], rows_bufs[j],
                gsems[j]).wait()

    for r in range(8):
        _, _, dt = _RELS[r]
        R = _RPW[dt]
        lo = sid * R
        half_len = src_refs[r].shape[0] // _NC
        base_e = cid * half_len
        n_chunks = half_len // _C
        n_src = m_refs[r].shape[0]

        def zero_body(j, _):
            for k in range(_KB):
                accs[k][pl.ds(j * _LANES, _LANES)] = zeros16
            return 0
        lax.fori_loop(0, R + 1, zero_body, 0)

        cclamp = lambda c: jnp.minimum(c, n_chunks - 1)

        def efire(c, sbuf, dbuf, esem, r=r, base_e=base_e):
            off = base_e + cclamp(c) * _C
            pltpu.async_copy(src_refs[r].at[pl.ds(off, _C)], sbuf, esem)
            pltpu.async_copy(dst_refs[r].at[pl.ds(off, _C)], dbuf, esem)

        def edrain(sbuf, dbuf, esem, r=r):
            pltpu.make_async_copy(src_refs[r].at[pl.ds(0, _C)], sbuf,
                                  esem).wait()
            pltpu.make_async_copy(dst_refs[r].at[pl.ds(0, _C)], dbuf,
                                  esem).wait()

        efire(0, srcA, dstA, esemA)
        efire(1, srcB, dstB, esemB)

        def cpair_body(cp, _, r=r, R=R, lo=lo):
            edrain(srcA, dstA, esemA)
            scan_and_merge(r, R, lo, srcA, dstA)
            efire(2 * cp + 2, srcA, dstA, esemA)
            edrain(srcB, dstB, esemB)
            scan_and_merge(r, R, lo, srcB, dstB)
            efire(2 * cp + 3, srcB, dstB, esemB)
            return 0

        lax.fori_loop(0, (n_chunks + 1) // 2, cpair_body, 0)
        edrain(srcA, dstA, esemA)
        edrain(srcB, dstB, esemB)

        wbase = (wid * _KB) * (R * _LANES)
        for k in range(_KB):
            pltpu.sync_copy(
                accs[k].at[pl.ds(0, R * _LANES)],
                out_refs[r].at[pl.ds(wbase + k * (R * _LANES), R * _LANES)])


def _segment_max_layer(ms, srcs, dsts):
    out_types = [
        jax.ShapeDtypeStruct((_NC * _NS * _KB * _RPW[_RELS[r][2]] * _LANES,),
                             jnp.float32)
        for r in range(8)
    ]
    mesh = plsc.VectorSubcoreMesh(core_axis_name="c", subcore_axis_name="s")
    fn = pl.kernel(
        _seg_body,
        out_type=out_types,
        mesh=mesh,
        scratch_types=(
            [pltpu.VMEM((_C,), jnp.int32)] * 4
            + [pltpu.VMEM((_C + 64,), jnp.int32)] * 2
            + [pltpu.VMEM((_G, H), jnp.float32)] * 2
            + [pltpu.VMEM(((_RMAX + 1) * _LANES,), jnp.float32)] * _KB
            + [pltpu.SemaphoreType.DMA] * 4
        ),
        compiler_params=pltpu.CompilerParams(needs_layout_passes=False),
    )
    outs = fn(*ms, *srcs, *dsts)
    res = []
    for r, o in enumerate(outs):
        n_dst = _NNODES[_RELS[r][2]]
        R = _RPW[_RELS[r][2]]
        o = o.reshape(_NS, _NC, _KB, R, _LANES).transpose(1, 0, 3, 2, 4)
        o = o.reshape(_NC, _NS * R, H)
        res.append(jnp.maximum(o[0], o[1])[:n_dst])
    return res


def _mm_relu_body(x_ref, w_ref, b_ref, o_ref):
    o_ref[...] = jax.nn.relu(
        jnp.dot(x_ref[...], w_ref[...], preferred_element_type=jnp.float32)
        + b_ref[...])


def _mm_relu(x, w, b, block_rows=0):
    n, k = x.shape
    ho = w.shape[1]
    if not block_rows or block_rows >= n:
        return pl.pallas_call(
            _mm_relu_body,
            out_shape=jax.ShapeDtypeStruct((n, ho), jnp.float32),
        )(x, w, b)
    assert n % block_rows == 0
    return pl.pallas_call(
        _mm_relu_body,
        grid=(n // block_rows,),
        in_specs=[
            pl.BlockSpec((block_rows, k), lambda i: (i, 0)),
            pl.BlockSpec((k, ho), lambda i: (0, 0)),
            pl.BlockSpec((ho,), lambda i: (0,)),
        ],
        out_specs=pl.BlockSpec((block_rows, ho), lambda i: (i, 0)),
        out_shape=jax.ShapeDtypeStruct((n, ho), jnp.float32),
    )(x, w, b)


def _sage_out_body(h_ref, n_ref, ws_ref, wn_ref, b_ref, o_ref):
    acc = None
    for j in range(n_ref.shape[0]):
        t = jax.nn.relu(
            jnp.dot(h_ref[...], ws_ref[j], preferred_element_type=jnp.float32)
            + jnp.dot(n_ref[j], wn_ref[j], preferred_element_type=jnp.float32)
            + b_ref[j])
        acc = t if acc is None else acc + t
    o_ref[...] = acc


def _sage_out(h_dt, neighs, Ws_sel, Wn_sel, bs_sel):
    n = h_dt.shape[0]
    return pl.pallas_call(
        _sage_out_body,
        out_shape=jax.ShapeDtypeStruct((n, H), jnp.float32),
    )(h_dt, jnp.stack(neighs), Ws_sel, Wn_sel, bs_sel)


def _bn_relu(z, g, b):
    m = jnp.mean(z, 0)
    v = jnp.mean((z - m) ** 2, 0)
    return jax.nn.relu((z - m) * lax.rsqrt(v + 1e-5) * g + b)


def _head_body(feat_ref, W1, b1, g1, be1, W2, b2, g2, be2, W3, b3, g3, be3,
               Wout, bout, out_ref):
    z = jnp.dot(feat_ref[...], W1[...], preferred_element_type=jnp.float32) + b1[...]
    o = _bn_relu(z, g1[...], be1[...])
    z = jnp.dot(o, W2[...], preferred_element_type=jnp.float32) + b2[...]
    o = _bn_relu(z, g2[...], be2[...])
    z = jnp.dot(o, W3[...], preferred_element_type=jnp.float32) + b3[...]
    o = _bn_relu(z, g3[...], be3[...])
    z = jnp.dot(o, Wout[...], preferred_element_type=jnp.float32) + bout[...]
    out_ref[...] = jax.nn.sigmoid(z)


def _head(feat, W1, b1, g1, be1, W2, b2, g2, be2, W3, b3, g3, be3, Wout, bout):
    B = feat.shape[0]
    return pl.pallas_call(
        _head_body,
        out_shape=jax.ShapeDtypeStruct((B, 1), jnp.float32),
    )(feat, W1, b1, g1, be1, W2, b2, g2, be2, W3, b3, g3, be3, Wout, bout)


_SRC_GROUPS = {"disease": [0, 1, 2], "drug": [3, 4, 6], "protein": [5, 7]}
_DST_GROUPS = {"drug": [0, 1, 6], "protein": [2, 7], "disease": [3, 4, 5]}


def _hgcn(h, srcs, dsts, Wp, bp, Ws, Wn, bs):
    ms = [None] * 8
    for st, ids in _SRC_GROUPS.items():
        wcat = jnp.concatenate([Wp[i] for i in ids], axis=1)
        bcat = jnp.concatenate([bp[i] for i in ids])
        mcat = _mm_relu(h[st], wcat, bcat)
        for j, i in enumerate(ids):
            ms[i] = mcat[:, j * H:(j + 1) * H]
    neighs = _segment_max_layer(ms, srcs, dsts)
    out = {}
    for dt, ids in _DST_GROUPS.items():
        out[dt] = _sage_out(h[dt], [neighs[i] for i in ids],
                            Ws[jnp.array(ids)], Wn[jnp.array(ids)],
                            bs[jnp.array(ids)])
    return out


def kernel(x_dr, x_p, finger_feats, seq_feats, disease_feat, e_d_t_dr, e_d_m_dr,
           e_d_p, e_dr_t_d, e_dr_m_d, e_p_d, e_DDI, e_PPI, W_fing, b_fing,
           W_seq, b_seq, W_dis, b_dis, Wp, bp, Ws, Wn, bs, W1, b1, g1, be1,
           W2, b2, g2, be2, W3, b3, g3, be3, Wout, bout):
    edges = [e_d_t_dr, e_d_m_dr, e_d_p, e_dr_t_d, e_dr_m_d, e_p_d, e_DDI,
             e_PPI]
    srcs, dsts = [], []
    for e in edges:
        ne = e.shape[1]
        half = ne // _NC
        pad = _rup(half, _C) - half
        s2 = e[0].reshape(_NC, half).astype(jnp.int32)
        d2 = e[1].reshape(_NC, half).astype(jnp.int32)
        srcs.append(jnp.pad(s2, ((0, 0), (0, pad))).reshape(-1))
        dsts.append(jnp.pad(d2, ((0, 0), (0, pad)),
                            constant_values=-1).reshape(-1))

    h_dr_f = _mm_relu(finger_feats, W_fing, b_fing, block_rows=2000)
    h_p_s = _mm_relu(seq_feats, W_seq, b_seq, block_rows=2000)
    h_d = _mm_relu(disease_feat, W_dis, b_dis)
    h0 = {"drug": h_dr_f, "protein": h_p_s, "disease": h_d}
    h1 = _hgcn(h0, srcs, dsts, Wp, bp, Ws, Wn, bs)
    h2 = _hgcn(h1, srcs, dsts, Wp, bp, Ws, Wn, bs)
    dr_new = jnp.concatenate([h_dr_f, h1["drug"], h2["drug"]], axis=1)
    p_new = jnp.concatenate([h_p_s, h1["protein"], h2["protein"]], axis=1)
    feat = jnp.concatenate([dr_new[x_dr], p_new[x_p]], axis=1)
    return _head(feat, W1, b1, g1, be1, W2, b2, g2, be2, W3, b3, g3, be3,
                 Wout, bout)
